# baseline (speedup 1.0000x reference)
import jax
import jax.numpy as jnp
import numpy as np
from jax import lax
from jax.experimental import pallas as pl
from jax.experimental.pallas import tpu as pltpu
from jax.experimental.pallas import tpu_sc as plsc

N_NODES = 10000
D = 128
N_EDGES = 320000

NUM_TILES = 32
FPT = D // NUM_TILES
PPT = FPT // 2
EC = 4000
NCHUNK = N_EDGES // EC

NINF_PAIR = int(np.array(0xFF80FF80, dtype=np.uint32).view(np.int32))
ONES_PAIR = int(np.array(0x3F803F80, dtype=np.uint32).view(np.int32))


def _sc_body(xt_hbm, src_hbm, dst_hbm, acc_hbm,
             xp0, xp1, ap0, ap1, srcv, dstv, sem0, sem1):
    cid = lax.axis_index("c")
    sid = lax.axis_index("s")
    wid = sid * 2 + cid
    seg = PPT * N_NODES

    xps = [xp0, xp1]
    aps = [ap0, ap1]
    sems = [sem0, sem1]

    for p in range(PPT):
        pltpu.sync_copy(
            xt_hbm.at[pl.ds(wid * seg + p * N_NODES, N_NODES)], xps[p])

    ninf16 = jnp.full((16,), NINF_PAIR, dtype=jnp.int32)

    def init_acc(r, carry):
        for p in range(PPT):
            aps[p][pl.ds(r * 16, 16)] = ninf16
        return carry

    lax.fori_loop(0, N_NODES // 16, init_acc, 0)

    def fire(ch, slot):
        ebase = ch * EC
        sem = sems[slot]
        pltpu.async_copy(src_hbm.at[pl.ds(ebase, EC)],
                         srcv.at[pl.ds(slot * EC, EC)], sem)
        pltpu.async_copy(dst_hbm.at[pl.ds(ebase, EC)],
                         dstv.at[pl.ds(slot * EC, EC)], sem)

    def drain(slot):
        sem = sems[slot]
        pltpu.make_async_copy(src_hbm.at[pl.ds(0, EC)],
                              srcv.at[pl.ds(slot * EC, EC)], sem).wait()
        pltpu.make_async_copy(dst_hbm.at[pl.ds(0, EC)],
                              dstv.at[pl.ds(slot * EC, EC)], sem).wait()

    def process(slot):
        base = slot * EC

        def step(j, resid):
            sv = srcv[pl.ds(base + j * 16, 16)]
            dv = dstv[pl.ds(base + j * 16, 16)]
            vals = [plsc.bitcast(plsc.load_gather(xps[p], [sv]),
                                 jnp.bfloat16) for p in range(PPT)]

            for p in range(PPT):
                cur = plsc.bitcast(plsc.load_gather(aps[p], [dv]),
                                   jnp.bfloat16)
                plsc.store_scatter(
                    aps[p], [dv],
                    plsc.bitcast(jnp.maximum(cur, vals[p]), jnp.int32))

            for p in range(PPT):
                back = plsc.bitcast(plsc.load_gather(aps[p], [dv]),
                                    jnp.bfloat16)
                okh = jnp.where(back >= vals[p],
                                jnp.bfloat16(1.0), jnp.bfloat16(0.0))
                pend = plsc.bitcast(okh, jnp.int32) != ONES_PAIR
                plsc.store_scatter(
                    aps[p], [dv],
                    plsc.bitcast(jnp.maximum(back, vals[p]), jnp.int32),
                    mask=pend)

            cnt, _ = plsc.scan_count(dv)
            return resid | (cnt >= 3).astype(jnp.int32)

        resid = lax.fori_loop(0, EC // 16, step,
                              jnp.zeros((16,), dtype=jnp.int32))
        nres = plsc.all_reduce_population_count(resid != 0)

        @pl.when(nres[0] > 0)
        def _():
            def fixstep(j, carry2):
                sv = srcv[pl.ds(base + j * 16, 16)]
                dv = dstv[pl.ds(base + j * 16, 16)]
                vals = [plsc.bitcast(plsc.load_gather(xps[p], [sv]),
                                     jnp.bfloat16) for p in range(PPT)]
                for _ in range(15):
                    for p in range(PPT):
                        back = plsc.bitcast(plsc.load_gather(aps[p], [dv]),
                                            jnp.bfloat16)
                        okh = jnp.where(back >= vals[p],
                                        jnp.bfloat16(1.0), jnp.bfloat16(0.0))
                        pend = plsc.bitcast(okh, jnp.int32) != ONES_PAIR
                        plsc.store_scatter(
                            aps[p], [dv],
                            plsc.bitcast(jnp.maximum(back, vals[p]),
                                         jnp.int32),
                            mask=pend)
                return carry2

            lax.fori_loop(0, EC // 16, fixstep, 0)

    fire(0, 0)

    def pair_body(i, carry):
        ch0 = pl.multiple_of(i * 2, 2)
        drain(0)
        fire(ch0 + 1, 1)
        process(0)
        drain(1)

        @pl.when(ch0 + 2 < NCHUNK)
        def _():
            fire(ch0 + 2, 0)

        process(1)
        return carry

    lax.fori_loop(0, NCHUNK // 2, pair_body, 0)

    for p in range(PPT):
        pltpu.sync_copy(
            aps[p], acc_hbm.at[pl.ds(wid * seg + p * N_NODES, N_NODES)])


def _segment_max_sc(xt, src, dst):
    mesh = plsc.VectorSubcoreMesh(core_axis_name="c", subcore_axis_name="s",
                                  num_cores=2, num_subcores=16)
    return pl.kernel(
        _sc_body,
        out_type=jax.ShapeDtypeStruct((NUM_TILES * PPT * N_NODES,),
                                      jnp.int32),
        mesh=mesh,
        scratch_types=[
            pltpu.VMEM((N_NODES,), jnp.int32),
            pltpu.VMEM((N_NODES,), jnp.int32),
            pltpu.VMEM((N_NODES,), jnp.int32),
            pltpu.VMEM((N_NODES,), jnp.int32),
            pltpu.VMEM((2 * EC,), jnp.int32),
            pltpu.VMEM((2 * EC,), jnp.int32),
            pltpu.SemaphoreType.DMA,
            pltpu.SemaphoreType.DMA,
        ],
        compiler_params=pltpu.CompilerParams(needs_layout_passes=False),
    )(xt, src, dst)


def _dense_body(x_ref, m_ref, w_ref, b_ref, o_ref):
    xb = x_ref[...]
    mb = m_ref[...].astype(jnp.float32)
    xj = jnp.where(jnp.isneginf(mb), 0.0, mb - xb)
    h = jnp.dot(xb, w_ref[0:D, :], preferred_element_type=jnp.float32)
    h = h + jnp.dot(xj, w_ref[D:2 * D, :], preferred_element_type=jnp.float32)
    o_ref[...] = jnp.maximum(h + b_ref[...], 0.0)


def _dense_tc(x, m, W, b):
    blk = 400
    grid = N_NODES // blk
    return pl.pallas_call(
        _dense_body,
        out_shape=jax.ShapeDtypeStruct((N_NODES, D), jnp.float32),
        grid=(grid,),
        in_specs=[
            pl.BlockSpec((blk, D), lambda i: (i, 0)),
            pl.BlockSpec((blk, D), lambda i: (i, 0)),
            pl.BlockSpec((2 * D, D), lambda i: (0, 0)),
            pl.BlockSpec((1, D), lambda i: (0, 0)),
        ],
        out_specs=pl.BlockSpec((blk, D), lambda i: (i, 0)),
    )(x, m, W, b)


def kernel(x, edge_index, W, b):
    src = edge_index[0].astype(jnp.int32)
    dst = edge_index[1].astype(jnp.int32)
    xb = x.astype(jnp.bfloat16)
    xp = lax.bitcast_convert_type(xb.reshape(N_NODES, D // 2, 2), jnp.int32)
    xt = xp.T.reshape(-1)
    acc = _segment_max_sc(xt, src, dst)
    mp = acc.reshape(D // 2, N_NODES).T
    m = lax.bitcast_convert_type(mp, jnp.bfloat16).reshape(N_NODES, D)
    return _dense_tc(x, m, W, b.reshape(1, D))


# speedup vs baseline: 1.0042x; 1.0042x over previous
import jax
import jax.numpy as jnp
import numpy as np
from jax import lax
from jax.experimental import pallas as pl
from jax.experimental.pallas import tpu as pltpu
from jax.experimental.pallas import tpu_sc as plsc

N_NODES = 10000
D = 128
N_EDGES = 320000

NUM_TILES = 32
FPT = D // NUM_TILES
PPT = FPT // 2
EC = 8000
NCHUNK = N_EDGES // EC

NINF_PAIR = int(np.array(0xFF80FF80, dtype=np.uint32).view(np.int32))
ONES_PAIR = int(np.array(0x3F803F80, dtype=np.uint32).view(np.int32))


def _sc_body(xt_hbm, src_hbm, dst_hbm, acc_hbm,
             xp0, xp1, ap0, ap1, srcv, dstv, sem0, sem1):
    cid = lax.axis_index("c")
    sid = lax.axis_index("s")
    wid = sid * 2 + cid
    seg = PPT * N_NODES

    xps = [xp0, xp1]
    aps = [ap0, ap1]
    sems = [sem0, sem1]

    for p in range(PPT):
        pltpu.sync_copy(
            xt_hbm.at[pl.ds(wid * seg + p * N_NODES, N_NODES)], xps[p])

    ninf16 = jnp.full((16,), NINF_PAIR, dtype=jnp.int32)

    def init_acc(r, carry):
        for p in range(PPT):
            aps[p][pl.ds(r * 16, 16)] = ninf16
        return carry

    lax.fori_loop(0, N_NODES // 16, init_acc, 0)

    def fire(ch, slot):
        ebase = ch * EC
        sem = sems[slot]
        pltpu.async_copy(src_hbm.at[pl.ds(ebase, EC)],
                         srcv.at[pl.ds(slot * EC, EC)], sem)
        pltpu.async_copy(dst_hbm.at[pl.ds(ebase, EC)],
                         dstv.at[pl.ds(slot * EC, EC)], sem)

    def drain(slot):
        sem = sems[slot]
        pltpu.make_async_copy(src_hbm.at[pl.ds(0, EC)],
                              srcv.at[pl.ds(slot * EC, EC)], sem).wait()
        pltpu.make_async_copy(dst_hbm.at[pl.ds(0, EC)],
                              dstv.at[pl.ds(slot * EC, EC)], sem).wait()

    def process(slot):
        base = slot * EC

        def step(j, resid):
            sv = srcv[pl.ds(base + j * 16, 16)]
            dv = dstv[pl.ds(base + j * 16, 16)]
            vals = [plsc.bitcast(plsc.load_gather(xps[p], [sv]),
                                 jnp.bfloat16) for p in range(PPT)]

            for p in range(PPT):
                cur = plsc.bitcast(plsc.load_gather(aps[p], [dv]),
                                   jnp.bfloat16)
                plsc.store_scatter(
                    aps[p], [dv],
                    plsc.bitcast(jnp.maximum(cur, vals[p]), jnp.int32))

            for p in range(PPT):
                back = plsc.bitcast(plsc.load_gather(aps[p], [dv]),
                                    jnp.bfloat16)
                okh = jnp.where(back >= vals[p],
                                jnp.bfloat16(1.0), jnp.bfloat16(0.0))
                pend = plsc.bitcast(okh, jnp.int32) != ONES_PAIR
                plsc.store_scatter(
                    aps[p], [dv],
                    plsc.bitcast(jnp.maximum(back, vals[p]), jnp.int32),
                    mask=pend)

            cnt, _ = plsc.scan_count(dv)
            return resid | (cnt >= 3).astype(jnp.int32)

        resid = lax.fori_loop(0, EC // 16, step,
                              jnp.zeros((16,), dtype=jnp.int32))
        nres = plsc.all_reduce_population_count(resid != 0)

        @pl.when(nres[0] > 0)
        def _():
            def fixstep(j, carry2):
                sv = srcv[pl.ds(base + j * 16, 16)]
                dv = dstv[pl.ds(base + j * 16, 16)]
                vals = [plsc.bitcast(plsc.load_gather(xps[p], [sv]),
                                     jnp.bfloat16) for p in range(PPT)]
                for _ in range(15):
                    for p in range(PPT):
                        back = plsc.bitcast(plsc.load_gather(aps[p], [dv]),
                                            jnp.bfloat16)
                        okh = jnp.where(back >= vals[p],
                                        jnp.bfloat16(1.0), jnp.bfloat16(0.0))
                        pend = plsc.bitcast(okh, jnp.int32) != ONES_PAIR
                        plsc.store_scatter(
                            aps[p], [dv],
                            plsc.bitcast(jnp.maximum(back, vals[p]),
                                         jnp.int32),
                            mask=pend)
                return carry2

            lax.fori_loop(0, EC // 16, fixstep, 0)

    fire(0, 0)

    def pair_body(i, carry):
        ch0 = pl.multiple_of(i * 2, 2)
        drain(0)
        fire(ch0 + 1, 1)
        process(0)
        drain(1)

        @pl.when(ch0 + 2 < NCHUNK)
        def _():
            fire(ch0 + 2, 0)

        process(1)
        return carry

    lax.fori_loop(0, NCHUNK // 2, pair_body, 0)

    for p in range(PPT):
        pltpu.sync_copy(
            aps[p], acc_hbm.at[pl.ds(wid * seg + p * N_NODES, N_NODES)])


def _segment_max_sc(xt, src, dst):
    mesh = plsc.VectorSubcoreMesh(core_axis_name="c", subcore_axis_name="s",
                                  num_cores=2, num_subcores=16)
    return pl.kernel(
        _sc_body,
        out_type=jax.ShapeDtypeStruct((NUM_TILES * PPT * N_NODES,),
                                      jnp.int32),
        mesh=mesh,
        scratch_types=[
            pltpu.VMEM((N_NODES,), jnp.int32),
            pltpu.VMEM((N_NODES,), jnp.int32),
            pltpu.VMEM((N_NODES,), jnp.int32),
            pltpu.VMEM((N_NODES,), jnp.int32),
            pltpu.VMEM((2 * EC,), jnp.int32),
            pltpu.VMEM((2 * EC,), jnp.int32),
            pltpu.SemaphoreType.DMA,
            pltpu.SemaphoreType.DMA,
        ],
        compiler_params=pltpu.CompilerParams(needs_layout_passes=False),
    )(xt, src, dst)


def _dense_body(x_ref, m_ref, w_ref, b_ref, o_ref):
    xb = x_ref[...]
    mb = m_ref[...].astype(jnp.float32)
    xj = jnp.where(jnp.isneginf(mb), 0.0, mb - xb)
    h = jnp.dot(xb, w_ref[0:D, :], preferred_element_type=jnp.float32)
    h = h + jnp.dot(xj, w_ref[D:2 * D, :], preferred_element_type=jnp.float32)
    o_ref[...] = jnp.maximum(h + b_ref[...], 0.0)


def _dense_tc(x, m, W, b):
    blk = 400
    grid = N_NODES // blk
    return pl.pallas_call(
        _dense_body,
        out_shape=jax.ShapeDtypeStruct((N_NODES, D), jnp.float32),
        grid=(grid,),
        in_specs=[
            pl.BlockSpec((blk, D), lambda i: (i, 0)),
            pl.BlockSpec((blk, D), lambda i: (i, 0)),
            pl.BlockSpec((2 * D, D), lambda i: (0, 0)),
            pl.BlockSpec((1, D), lambda i: (0, 0)),
        ],
        out_specs=pl.BlockSpec((blk, D), lambda i: (i, 0)),
    )(x, m, W, b)


def kernel(x, edge_index, W, b):
    src = edge_index[0].astype(jnp.int32)
    dst = edge_index[1].astype(jnp.int32)
    xb = x.astype(jnp.bfloat16)
    xp = lax.bitcast_convert_type(xb.reshape(N_NODES, D // 2, 2), jnp.int32)
    xt = xp.T.reshape(-1)
    acc = _segment_max_sc(xt, src, dst)
    mp = acc.reshape(D // 2, N_NODES).T
    m = lax.bitcast_convert_type(mp, jnp.bfloat16).reshape(N_NODES, D)
    return _dense_tc(x, m, W, b.reshape(1, D))


# speedup vs baseline: 1.0120x; 1.0077x over previous
import jax
import jax.numpy as jnp
import numpy as np
from jax import lax
from jax.experimental import pallas as pl
from jax.experimental.pallas import tpu as pltpu
from jax.experimental.pallas import tpu_sc as plsc

N_NODES = 10000
D = 128
N_EDGES = 320000

NUM_TILES = 32
FPT = D // NUM_TILES
PPT = FPT // 2
EC = 8000
NCHUNK = N_EDGES // EC

NINF_PAIR = int(np.array(0xFF80FF80, dtype=np.uint32).view(np.int32))
ONES_PAIR = int(np.array(0x3F803F80, dtype=np.uint32).view(np.int32))


def _sc_body(xt_hbm, src_hbm, dst_hbm, acc_hbm,
             xp0, xp1, ap0, ap1, srcv, dstv, sem0, sem1):
    cid = lax.axis_index("c")
    sid = lax.axis_index("s")
    wid = sid * 2 + cid
    seg = PPT * N_NODES

    xps = [xp0, xp1]
    aps = [ap0, ap1]
    sems = [sem0, sem1]

    for p in range(PPT):
        pltpu.sync_copy(
            xt_hbm.at[pl.ds(wid * seg + p * N_NODES, N_NODES)], xps[p])

    ninf16 = jnp.full((16,), NINF_PAIR, dtype=jnp.int32)

    def init_acc(r, carry):
        for p in range(PPT):
            aps[p][pl.ds(r * 16, 16)] = ninf16
        return carry

    lax.fori_loop(0, N_NODES // 16, init_acc, 0)

    def fire(ch, slot):
        ebase = ch * EC
        sem = sems[slot]
        pltpu.async_copy(src_hbm.at[pl.ds(ebase, EC)],
                         srcv.at[pl.ds(slot * EC, EC)], sem)
        pltpu.async_copy(dst_hbm.at[pl.ds(ebase, EC)],
                         dstv.at[pl.ds(slot * EC, EC)], sem)

    def drain(slot):
        sem = sems[slot]
        pltpu.make_async_copy(src_hbm.at[pl.ds(0, EC)],
                              srcv.at[pl.ds(slot * EC, EC)], sem).wait()
        pltpu.make_async_copy(dst_hbm.at[pl.ds(0, EC)],
                              dstv.at[pl.ds(slot * EC, EC)], sem).wait()

    def process(slot):
        base = slot * EC

        def one_vec(off, resid):
            sv = srcv[pl.ds(off, 16)]
            dv = dstv[pl.ds(off, 16)]
            vals = [plsc.bitcast(plsc.load_gather(xps[p], [sv]),
                                 jnp.bfloat16) for p in range(PPT)]

            for p in range(PPT):
                cur = plsc.bitcast(plsc.load_gather(aps[p], [dv]),
                                   jnp.bfloat16)
                plsc.store_scatter(
                    aps[p], [dv],
                    plsc.bitcast(jnp.maximum(cur, vals[p]), jnp.int32))

            for p in range(PPT):
                back = plsc.bitcast(plsc.load_gather(aps[p], [dv]),
                                    jnp.bfloat16)
                okh = jnp.where(back >= vals[p],
                                jnp.bfloat16(1.0), jnp.bfloat16(0.0))
                pend = plsc.bitcast(okh, jnp.int32) != ONES_PAIR
                plsc.store_scatter(
                    aps[p], [dv],
                    plsc.bitcast(jnp.maximum(back, vals[p]), jnp.int32),
                    mask=pend)

            cnt, _ = plsc.scan_count(dv)
            return resid | (cnt >= 3).astype(jnp.int32)

        def step(j, resid):
            off = base + j * 32
            resid = one_vec(off, resid)
            return one_vec(off + 16, resid)

        resid = lax.fori_loop(0, EC // 32, step,
                              jnp.zeros((16,), dtype=jnp.int32))
        nres = plsc.all_reduce_population_count(resid != 0)

        @pl.when(nres[0] > 0)
        def _():
            def fixstep(j, carry2):
                sv = srcv[pl.ds(base + j * 16, 16)]
                dv = dstv[pl.ds(base + j * 16, 16)]
                vals = [plsc.bitcast(plsc.load_gather(xps[p], [sv]),
                                     jnp.bfloat16) for p in range(PPT)]
                for _ in range(15):
                    for p in range(PPT):
                        back = plsc.bitcast(plsc.load_gather(aps[p], [dv]),
                                            jnp.bfloat16)
                        okh = jnp.where(back >= vals[p],
                                        jnp.bfloat16(1.0), jnp.bfloat16(0.0))
                        pend = plsc.bitcast(okh, jnp.int32) != ONES_PAIR
                        plsc.store_scatter(
                            aps[p], [dv],
                            plsc.bitcast(jnp.maximum(back, vals[p]),
                                         jnp.int32),
                            mask=pend)
                return carry2

            lax.fori_loop(0, EC // 16, fixstep, 0)

    fire(0, 0)

    def pair_body(i, carry):
        ch0 = pl.multiple_of(i * 2, 2)
        drain(0)
        fire(ch0 + 1, 1)
        process(0)
        drain(1)

        @pl.when(ch0 + 2 < NCHUNK)
        def _():
            fire(ch0 + 2, 0)

        process(1)
        return carry

    lax.fori_loop(0, NCHUNK // 2, pair_body, 0)

    for p in range(PPT):
        pltpu.sync_copy(
            aps[p], acc_hbm.at[pl.ds(wid * seg + p * N_NODES, N_NODES)])


def _segment_max_sc(xt, src, dst):
    mesh = plsc.VectorSubcoreMesh(core_axis_name="c", subcore_axis_name="s",
                                  num_cores=2, num_subcores=16)
    return pl.kernel(
        _sc_body,
        out_type=jax.ShapeDtypeStruct((NUM_TILES * PPT * N_NODES,),
                                      jnp.int32),
        mesh=mesh,
        scratch_types=[
            pltpu.VMEM((N_NODES,), jnp.int32),
            pltpu.VMEM((N_NODES,), jnp.int32),
            pltpu.VMEM((N_NODES,), jnp.int32),
            pltpu.VMEM((N_NODES,), jnp.int32),
            pltpu.VMEM((2 * EC,), jnp.int32),
            pltpu.VMEM((2 * EC,), jnp.int32),
            pltpu.SemaphoreType.DMA,
            pltpu.SemaphoreType.DMA,
        ],
        compiler_params=pltpu.CompilerParams(needs_layout_passes=False),
    )(xt, src, dst)


def _dense_body(x_ref, m_ref, w_ref, b_ref, o_ref):
    xb = x_ref[...]
    mb = m_ref[...].astype(jnp.float32)
    xj = jnp.where(jnp.isneginf(mb), 0.0, mb - xb)
    h = jnp.dot(xb, w_ref[0:D, :], preferred_element_type=jnp.float32)
    h = h + jnp.dot(xj, w_ref[D:2 * D, :], preferred_element_type=jnp.float32)
    o_ref[...] = jnp.maximum(h + b_ref[...], 0.0)


def _dense_tc(x, m, W, b):
    blk = 400
    grid = N_NODES // blk
    return pl.pallas_call(
        _dense_body,
        out_shape=jax.ShapeDtypeStruct((N_NODES, D), jnp.float32),
        grid=(grid,),
        in_specs=[
            pl.BlockSpec((blk, D), lambda i: (i, 0)),
            pl.BlockSpec((blk, D), lambda i: (i, 0)),
            pl.BlockSpec((2 * D, D), lambda i: (0, 0)),
            pl.BlockSpec((1, D), lambda i: (0, 0)),
        ],
        out_specs=pl.BlockSpec((blk, D), lambda i: (i, 0)),
    )(x, m, W, b)


def kernel(x, edge_index, W, b):
    src = edge_index[0].astype(jnp.int32)
    dst = edge_index[1].astype(jnp.int32)
    xb = x.astype(jnp.bfloat16)
    xp = lax.bitcast_convert_type(xb.reshape(N_NODES, D // 2, 2), jnp.int32)
    xt = xp.T.reshape(-1)
    acc = _segment_max_sc(xt, src, dst)
    mp = acc.reshape(D // 2, N_NODES).T
    m = lax.bitcast_convert_type(mp, jnp.bfloat16).reshape(N_NODES, D)
    return _dense_tc(x, m, W, b.reshape(1, D))


# speedup vs baseline: 1.0835x; 1.0707x over previous
import jax
import jax.numpy as jnp
import numpy as np
from jax import lax
from jax.experimental import pallas as pl
from jax.experimental.pallas import tpu as pltpu
from jax.experimental.pallas import tpu_sc as plsc

N_NODES = 10000
D = 128
N_EDGES = 320000

NUM_TILES = 32
FPT = D // NUM_TILES
PPT = FPT // 2
EC = 8000
NCHUNK = N_EDGES // EC

NINF_PAIR = int(np.array(0xFF80FF80, dtype=np.uint32).view(np.int32))
ONES_PAIR = int(np.array(0x3F803F80, dtype=np.uint32).view(np.int32))


def _sc_body(xt_hbm, src_hbm, dst_hbm, acc_hbm,
             xp0, xp1, ap0, ap1, srcv, dstv, sem0, sem1):
    cid = lax.axis_index("c")
    sid = lax.axis_index("s")
    wid = sid * 2 + cid
    seg = PPT * N_NODES

    xps = [xp0, xp1]
    aps = [ap0, ap1]
    sems = [sem0, sem1]

    for p in range(PPT):
        pltpu.sync_copy(
            xt_hbm.at[pl.ds(wid * seg + p * N_NODES, N_NODES)], xps[p])

    ninf16 = jnp.full((16,), NINF_PAIR, dtype=jnp.int32)

    def init_acc(r, carry):
        for p in range(PPT):
            aps[p][pl.ds(r * 16, 16)] = ninf16
        return carry

    lax.fori_loop(0, N_NODES // 16, init_acc, 0)

    def fire(ch, slot):
        ebase = ch * EC
        sem = sems[slot]
        pltpu.async_copy(src_hbm.at[pl.ds(ebase, EC)],
                         srcv.at[pl.ds(slot * EC, EC)], sem)
        pltpu.async_copy(dst_hbm.at[pl.ds(ebase, EC)],
                         dstv.at[pl.ds(slot * EC, EC)], sem)

    def drain(slot):
        sem = sems[slot]
        pltpu.make_async_copy(src_hbm.at[pl.ds(0, EC)],
                              srcv.at[pl.ds(slot * EC, EC)], sem).wait()
        pltpu.make_async_copy(dst_hbm.at[pl.ds(0, EC)],
                              dstv.at[pl.ds(slot * EC, EC)], sem).wait()

    def process(slot):
        base = slot * EC

        def one_vec(off, resid):
            sv = srcv[pl.ds(off, 16)]
            dv = dstv[pl.ds(off, 16)]
            vals = [plsc.bitcast(plsc.load_gather(xps[p], [sv]),
                                 jnp.bfloat16) for p in range(PPT)]
            cnt, lastm = plsc.scan_count(dv)

            for p in range(PPT):
                cur = plsc.bitcast(plsc.load_gather(aps[p], [dv]),
                                   jnp.bfloat16)
                plsc.store_scatter(
                    aps[p], [dv],
                    plsc.bitcast(jnp.maximum(cur, vals[p]), jnp.int32),
                    mask=lastm)

            nlast = ~lastm
            for p in range(PPT):
                back = plsc.bitcast(plsc.load_gather(aps[p], [dv]),
                                    jnp.bfloat16)
                plsc.store_scatter(
                    aps[p], [dv],
                    plsc.bitcast(jnp.maximum(back, vals[p]), jnp.int32),
                    mask=nlast)

            return resid | (cnt >= 3).astype(jnp.int32)

        def step(j, resid):
            off = base + j * 32
            resid = one_vec(off, resid)
            return one_vec(off + 16, resid)

        resid = lax.fori_loop(0, EC // 32, step,
                              jnp.zeros((16,), dtype=jnp.int32))
        nres = plsc.all_reduce_population_count(resid != 0)

        @pl.when(nres[0] > 0)
        def _():
            def fixstep(j, carry2):
                sv = srcv[pl.ds(base + j * 16, 16)]
                dv = dstv[pl.ds(base + j * 16, 16)]
                vals = [plsc.bitcast(plsc.load_gather(xps[p], [sv]),
                                     jnp.bfloat16) for p in range(PPT)]
                for _ in range(15):
                    for p in range(PPT):
                        back = plsc.bitcast(plsc.load_gather(aps[p], [dv]),
                                            jnp.bfloat16)
                        okh = jnp.where(back >= vals[p],
                                        jnp.bfloat16(1.0), jnp.bfloat16(0.0))
                        pend = plsc.bitcast(okh, jnp.int32) != ONES_PAIR
                        plsc.store_scatter(
                            aps[p], [dv],
                            plsc.bitcast(jnp.maximum(back, vals[p]),
                                         jnp.int32),
                            mask=pend)
                return carry2

            lax.fori_loop(0, EC // 16, fixstep, 0)

    fire(0, 0)

    def pair_body(i, carry):
        ch0 = pl.multiple_of(i * 2, 2)
        drain(0)
        fire(ch0 + 1, 1)
        process(0)
        drain(1)

        @pl.when(ch0 + 2 < NCHUNK)
        def _():
            fire(ch0 + 2, 0)

        process(1)
        return carry

    lax.fori_loop(0, NCHUNK // 2, pair_body, 0)

    for p in range(PPT):
        pltpu.sync_copy(
            aps[p], acc_hbm.at[pl.ds(wid * seg + p * N_NODES, N_NODES)])


def _segment_max_sc(xt, src, dst):
    mesh = plsc.VectorSubcoreMesh(core_axis_name="c", subcore_axis_name="s",
                                  num_cores=2, num_subcores=16)
    return pl.kernel(
        _sc_body,
        out_type=jax.ShapeDtypeStruct((NUM_TILES * PPT * N_NODES,),
                                      jnp.int32),
        mesh=mesh,
        scratch_types=[
            pltpu.VMEM((N_NODES,), jnp.int32),
            pltpu.VMEM((N_NODES,), jnp.int32),
            pltpu.VMEM((N_NODES,), jnp.int32),
            pltpu.VMEM((N_NODES,), jnp.int32),
            pltpu.VMEM((2 * EC,), jnp.int32),
            pltpu.VMEM((2 * EC,), jnp.int32),
            pltpu.SemaphoreType.DMA,
            pltpu.SemaphoreType.DMA,
        ],
        compiler_params=pltpu.CompilerParams(needs_layout_passes=False),
    )(xt, src, dst)


def _dense_body(x_ref, m_ref, w_ref, b_ref, o_ref):
    xb = x_ref[...]
    mb = m_ref[...].astype(jnp.float32)
    xj = jnp.where(jnp.isneginf(mb), 0.0, mb - xb)
    h = jnp.dot(xb, w_ref[0:D, :], preferred_element_type=jnp.float32)
    h = h + jnp.dot(xj, w_ref[D:2 * D, :], preferred_element_type=jnp.float32)
    o_ref[...] = jnp.maximum(h + b_ref[...], 0.0)


def _dense_tc(x, m, W, b):
    blk = 400
    grid = N_NODES // blk
    return pl.pallas_call(
        _dense_body,
        out_shape=jax.ShapeDtypeStruct((N_NODES, D), jnp.float32),
        grid=(grid,),
        in_specs=[
            pl.BlockSpec((blk, D), lambda i: (i, 0)),
            pl.BlockSpec((blk, D), lambda i: (i, 0)),
            pl.BlockSpec((2 * D, D), lambda i: (0, 0)),
            pl.BlockSpec((1, D), lambda i: (0, 0)),
        ],
        out_specs=pl.BlockSpec((blk, D), lambda i: (i, 0)),
    )(x, m, W, b)


def kernel(x, edge_index, W, b):
    src = edge_index[0].astype(jnp.int32)
    dst = edge_index[1].astype(jnp.int32)
    xb = x.astype(jnp.bfloat16)
    xp = lax.bitcast_convert_type(xb.reshape(N_NODES, D // 2, 2), jnp.int32)
    xt = xp.T.reshape(-1)
    acc = _segment_max_sc(xt, src, dst)
    mp = acc.reshape(D // 2, N_NODES).T
    m = lax.bitcast_convert_type(mp, jnp.bfloat16).reshape(N_NODES, D)
    return _dense_tc(x, m, W, b.reshape(1, D))
